# baseline (device time: 204642 ns/iter reference)
import jax
import jax.numpy as jnp
from jax import lax
from jax.experimental import pallas as pl
from jax.experimental.pallas import tpu as pltpu

N_DEV = 16


def kernel(A, B):
    m, k = A.shape
    k2, n = B.shape

    def body(a_ref, b_ref, out_ref, comm_ref, send_sems, recv_sems):
        my_pos = lax.axis_index("i")
        left = (my_pos - 1) % N_DEV
        right = (my_pos + 1) % N_DEV

        barrier_sem = pltpu.get_barrier_semaphore()
        for nbr in (left, right):
            pl.semaphore_signal(
                barrier_sem, inc=1,
                device_id=(nbr,), device_id_type=pl.DeviceIdType.MESH,
            )
        pl.semaphore_wait(barrier_sem, 2)

        partial = jnp.dot(a_ref[:, :], b_ref[:, :],
                          preferred_element_type=jnp.float32)
        comm_ref[0, :, :] = partial
        acc = partial

        for h in range(N_DEV - 1):
            rdma = pltpu.make_async_remote_copy(
                src_ref=comm_ref.at[h],
                dst_ref=comm_ref.at[h + 1],
                send_sem=send_sems.at[h],
                recv_sem=recv_sems.at[h],
                device_id=(right,),
                device_id_type=pl.DeviceIdType.MESH,
            )
            rdma.start()
            rdma.wait()
            acc = acc + comm_ref[h + 1, :, :]

        z = acc
        out_ref[:, :] = 0.5 * z * (
            1.0 + jnp.tanh(0.7978845608 * (z + 0.044715 * z * z * z))
        )

    return pl.pallas_call(
        body,
        out_shape=jax.ShapeDtypeStruct((m, n), jnp.float32),
        in_specs=[
            pl.BlockSpec(memory_space=pltpu.VMEM),
            pl.BlockSpec(memory_space=pltpu.VMEM),
        ],
        out_specs=pl.BlockSpec(memory_space=pltpu.VMEM),
        scratch_shapes=[
            pltpu.VMEM((N_DEV, m, n), jnp.float32),
            pltpu.SemaphoreType.DMA((N_DEV - 1,)),
            pltpu.SemaphoreType.DMA((N_DEV - 1,)),
        ],
        compiler_params=pltpu.CompilerParams(collective_id=0),
    )(A, B)


# device time: 31006 ns/iter; 6.6001x vs baseline; 6.6001x over previous
import jax
import jax.numpy as jnp
from jax import lax
from jax.experimental import pallas as pl
from jax.experimental.pallas import tpu as pltpu

N_DEV = 16


def kernel(A, B):
    m, k = A.shape
    k2, n = B.shape
    rows = m // N_DEV

    def body(a_ref, b_ref, out_ref, pbuf, rs_buf,
             send1, recv1, send2, recv2):
        my_pos = lax.axis_index("i")

        barrier_sem = pltpu.get_barrier_semaphore()
        for s in range(1, N_DEV):
            pl.semaphore_signal(
                barrier_sem, inc=1,
                device_id=((my_pos + s) % N_DEV,),
                device_id_type=pl.DeviceIdType.MESH,
            )
        pl.semaphore_wait(barrier_sem, N_DEV - 1)

        pbuf[:, :] = jnp.dot(a_ref[:, :], b_ref[:, :],
                             preferred_element_type=jnp.float32)

        phase1 = []
        for s in range(1, N_DEV):
            tgt = (my_pos + s) % N_DEV
            rdma = pltpu.make_async_remote_copy(
                src_ref=pbuf.at[pl.ds(tgt * rows, rows)],
                dst_ref=rs_buf.at[my_pos],
                send_sem=send1.at[tgt],
                recv_sem=recv1.at[my_pos],
                device_id=(tgt,),
                device_id_type=pl.DeviceIdType.MESH,
            )
            rdma.start()
            phase1.append(rdma)

        rs_buf[my_pos, :, :] = pbuf[pl.ds(my_pos * rows, rows), :]
        for s in range(1, N_DEV):
            src = (my_pos - s) % N_DEV
            pltpu.make_async_remote_copy(
                src_ref=rs_buf.at[src],
                dst_ref=rs_buf.at[src],
                send_sem=send1.at[src],
                recv_sem=recv1.at[src],
                device_id=(src,),
                device_id_type=pl.DeviceIdType.MESH,
            ).wait_recv()

        z = jnp.sum(rs_buf[:, :, :], axis=0)
        g = 0.5 * z * (1.0 + jnp.tanh(0.7978845608 * (z + 0.044715 * z * z * z)))
        out_ref[pl.ds(my_pos * rows, rows), :] = g

        phase2 = []
        for s in range(1, N_DEV):
            tgt = (my_pos + s) % N_DEV
            rdma = pltpu.make_async_remote_copy(
                src_ref=out_ref.at[pl.ds(my_pos * rows, rows)],
                dst_ref=out_ref.at[pl.ds(my_pos * rows, rows)],
                send_sem=send2.at[tgt],
                recv_sem=recv2.at[my_pos],
                device_id=(tgt,),
                device_id_type=pl.DeviceIdType.MESH,
            )
            rdma.start()
            phase2.append(rdma)

        for s in range(1, N_DEV):
            src = (my_pos - s) % N_DEV
            pltpu.make_async_remote_copy(
                src_ref=out_ref.at[pl.ds(src * rows, rows)],
                dst_ref=out_ref.at[pl.ds(src * rows, rows)],
                send_sem=send2.at[src],
                recv_sem=recv2.at[src],
                device_id=(src,),
                device_id_type=pl.DeviceIdType.MESH,
            ).wait_recv()

        for rdma in phase1 + phase2:
            rdma.wait_send()

    return pl.pallas_call(
        body,
        out_shape=jax.ShapeDtypeStruct((m, n), jnp.float32),
        in_specs=[
            pl.BlockSpec(memory_space=pltpu.VMEM),
            pl.BlockSpec(memory_space=pltpu.VMEM),
        ],
        out_specs=pl.BlockSpec(memory_space=pltpu.VMEM),
        scratch_shapes=[
            pltpu.VMEM((m, n), jnp.float32),
            pltpu.VMEM((N_DEV, rows, n), jnp.float32),
            pltpu.SemaphoreType.DMA((N_DEV,)),
            pltpu.SemaphoreType.DMA((N_DEV,)),
            pltpu.SemaphoreType.DMA((N_DEV,)),
            pltpu.SemaphoreType.DMA((N_DEV,)),
        ],
        compiler_params=pltpu.CompilerParams(collective_id=0),
    )(A, B)


# device time: 22563 ns/iter; 9.0698x vs baseline; 1.3742x over previous
import jax
import jax.numpy as jnp
from jax import lax
from jax.experimental import pallas as pl
from jax.experimental.pallas import tpu as pltpu

N_DEV = 16


def kernel(A, B):
    m, k = A.shape
    k2, n = B.shape
    rows = m // N_DEV

    def body(a_ref, b_ref, out_ref, pbuf16, rs16, ag16,
             send1, recv1, send2, recv2):
        my_pos = lax.axis_index("i")

        barrier_sem = pltpu.get_barrier_semaphore()
        for s in range(1, N_DEV):
            pl.semaphore_signal(
                barrier_sem, inc=1,
                device_id=((my_pos + s) % N_DEV,),
                device_id_type=pl.DeviceIdType.MESH,
            )

        partial = jnp.dot(a_ref[:, :], b_ref[:, :],
                          preferred_element_type=jnp.float32)
        pbuf16[:, :] = partial.astype(jnp.bfloat16)

        pl.semaphore_wait(barrier_sem, N_DEV - 1)

        phase1 = []
        for s in range(1, N_DEV):
            tgt = (my_pos + s) % N_DEV
            rdma = pltpu.make_async_remote_copy(
                src_ref=pbuf16.at[pl.ds(tgt * rows, rows)],
                dst_ref=rs16.at[my_pos],
                send_sem=send1.at[tgt],
                recv_sem=recv1.at[my_pos],
                device_id=(tgt,),
                device_id_type=pl.DeviceIdType.MESH,
            )
            rdma.start()
            phase1.append(rdma)

        rs16[my_pos, :, :] = pbuf16[pl.ds(my_pos * rows, rows), :]
        for s in range(1, N_DEV):
            src = (my_pos - s) % N_DEV
            pltpu.make_async_remote_copy(
                src_ref=rs16.at[src],
                dst_ref=rs16.at[src],
                send_sem=send1.at[src],
                recv_sem=recv1.at[src],
                device_id=(src,),
                device_id_type=pl.DeviceIdType.MESH,
            ).wait_recv()

        z = jnp.sum(rs16[:, :, :].astype(jnp.float32), axis=0)
        g = 0.5 * z * (1.0 + jnp.tanh(0.7978845608 * (z + 0.044715 * z * z * z)))
        ag16[my_pos, :, :] = g.astype(jnp.bfloat16)

        phase2 = []
        for s in range(1, N_DEV):
            tgt = (my_pos + s) % N_DEV
            rdma = pltpu.make_async_remote_copy(
                src_ref=ag16.at[my_pos],
                dst_ref=ag16.at[my_pos],
                send_sem=send2.at[tgt],
                recv_sem=recv2.at[my_pos],
                device_id=(tgt,),
                device_id_type=pl.DeviceIdType.MESH,
            )
            rdma.start()
            phase2.append(rdma)

        for s in range(1, N_DEV):
            src = (my_pos - s) % N_DEV
            pltpu.make_async_remote_copy(
                src_ref=ag16.at[src],
                dst_ref=ag16.at[src],
                send_sem=send2.at[src],
                recv_sem=recv2.at[src],
                device_id=(src,),
                device_id_type=pl.DeviceIdType.MESH,
            ).wait_recv()

        out_ref[:, :] = jnp.reshape(
            ag16[:, :, :], (m, n)
        ).astype(jnp.float32)

        for rdma in phase1 + phase2:
            rdma.wait_send()

    return pl.pallas_call(
        body,
        out_shape=jax.ShapeDtypeStruct((m, n), jnp.float32),
        in_specs=[
            pl.BlockSpec(memory_space=pltpu.VMEM),
            pl.BlockSpec(memory_space=pltpu.VMEM),
        ],
        out_specs=pl.BlockSpec(memory_space=pltpu.VMEM),
        scratch_shapes=[
            pltpu.VMEM((m, n), jnp.bfloat16),
            pltpu.VMEM((N_DEV, rows, n), jnp.bfloat16),
            pltpu.VMEM((N_DEV, rows, n), jnp.bfloat16),
            pltpu.SemaphoreType.DMA((N_DEV,)),
            pltpu.SemaphoreType.DMA((N_DEV,)),
            pltpu.SemaphoreType.DMA((N_DEV,)),
            pltpu.SemaphoreType.DMA((N_DEV,)),
        ],
        compiler_params=pltpu.CompilerParams(collective_id=0),
    )(A, B)
